# Initial kernel scaffold; baseline (speedup 1.0000x reference)
import jax
import jax.numpy as jnp
from jax.experimental import pallas as pl


def kernel(x, state, edge_index, edge_weight, W, attn_src, attn_dst):
    raise NotImplementedError("write your pallas kernel here")


# speedup vs baseline: 10.2234x; 10.2234x over previous
import functools

import jax
import jax.numpy as jnp
from jax import lax
from jax.experimental import pallas as pl
from jax.experimental.pallas import tpu as pltpu
from jax.experimental.pallas import tpu_sc as plsc

N = 10000
E = 320000
D = 128
OUT = 128
H = 4
HD = 32
NEG_SLOPE = 0.2

NC = 2
NS = 16
NW = NC * NS
ET = E // NW
C = 80
CB = 25
NB = ET // (C * CB)
NT = N // NS
ACCW = 136
BN = 1000


def _tc_head(x_ref, st_ref, w1_ref, w2_ref, at_ref, h_ref, s_ref):
    h = jnp.dot(x_ref[...], w1_ref[...], preferred_element_type=jnp.float32)
    h = h + jnp.dot(st_ref[...], w2_ref[...], preferred_element_type=jnp.float32)
    h_ref[...] = h
    s_ref[...] = jnp.dot(h, at_ref[...], preferred_element_type=jnp.float32)


def _tc_tail(p_ref, h_ref, o_ref):
    acc = p_ref[0] + p_ref[1]
    cols = []
    for hh in range(H):
        num = acc[:, hh * HD:(hh + 1) * HD]
        den = acc[:, 128 + hh][:, None] + 1e-12
        cols.append(num / den)
    o = jnp.concatenate(cols, axis=1) + h_ref[...]
    o_ref[...] = jnp.where(o > 0, o, jnp.exp(jnp.minimum(o, 0.0)) - 1.0)


def _splat(val):
    return jnp.full((16,), val, jnp.int32)


def _sc_body(h_hbm, stab_hbm, src_hbm, dst_hbm, out_hbm,
             acc_sh, sidx, didx, ssrc, sdst, hrows, msg, zbuf, sem):
    c = lax.axis_index("c")
    s = lax.axis_index("s")
    gid = c * NS + s

    zero16 = jnp.zeros((16,), jnp.float32)
    for r in range(5):
        for off in list(range(0, ACCW - 16, 16)) + [ACCW - 16]:
            zbuf[r, pl.ds(off, 16)] = zero16

    @pl.loop(0, NT // 5)
    def _zero(i):
        pltpu.sync_copy(zbuf, acc_sh.at[pl.ds(s * NT + i * 5, 5)])

    @pl.loop(0, C)
    def _ztail(i):
        msg[i, pl.ds(120, 16)] = zero16

    plsc.subcore_barrier()

    @pl.loop(0, NB)
    def _block(b):
        row0 = gid * (NB * CB) + b * CB
        pltpu.sync_copy(src_hbm.at[pl.ds(row0, CB)], sidx)
        pltpu.sync_copy(dst_hbm.at[pl.ds(row0, CB)], didx)

        @pl.loop(0, CB)
        def _chunk(j):
            a1 = pltpu.async_copy(stab_hbm.at[sidx.at[j]], ssrc, sem)
            a2 = pltpu.async_copy(stab_hbm.at[didx.at[j]], sdst, sem)
            a3 = pltpu.async_copy(h_hbm.at[sidx.at[j]], hrows, sem)
            a1.wait()
            a2.wait()
            a3.wait()
            for g in range(C // 16):
                ev = lax.iota(jnp.int32, 16) + g * 16
                for hh in range(H):
                    s1 = plsc.load_gather(ssrc, [ev, _splat(hh)])
                    s2 = plsc.load_gather(sdst, [ev, _splat(4 + hh)])
                    logit = s1 + s2
                    logit = jnp.where(logit >= 0, logit, NEG_SLOPE * logit)
                    w = jnp.exp(logit)
                    plsc.store_scatter(msg, [ev, _splat(128 + hh)], w)

            @pl.loop(0, C)
            def _row(i):
                ii = jnp.zeros((16,), jnp.int32) + i
                for hh in range(H):
                    wv = plsc.load_gather(msg, [ii, _splat(128 + hh)])
                    for q in range(2):
                        off = hh * HD + q * 16
                        msg[i, pl.ds(off, 16)] = hrows[i, pl.ds(off, 16)] * wv

            pltpu.sync_copy(msg, acc_sh.at[didx.at[j]], add=True)

    plsc.subcore_barrier()
    pltpu.sync_copy(acc_sh.at[pl.ds(s * NT, NT)],
                    out_hbm.at[c, pl.ds(s * NT, NT)])


@jax.jit
def kernel(x, state, edge_index, edge_weight, W, attn_src, attn_dst):
    del edge_weight
    eye = jnp.eye(H, dtype=jnp.float32)
    a_src = (attn_src[:, :, None] * eye[:, None, :]).reshape(OUT, H)
    a_dst = (attn_dst[:, :, None] * eye[:, None, :]).reshape(OUT, H)
    at = jnp.concatenate([a_src, a_dst], axis=1)
    w1 = W[:D]
    w2 = W[D:]

    h, stab = pl.pallas_call(
        _tc_head,
        grid=(N // BN,),
        in_specs=[
            pl.BlockSpec((BN, D), lambda i: (i, 0)),
            pl.BlockSpec((BN, D), lambda i: (i, 0)),
            pl.BlockSpec((D, OUT), lambda i: (0, 0)),
            pl.BlockSpec((D, OUT), lambda i: (0, 0)),
            pl.BlockSpec((OUT, 2 * H), lambda i: (0, 0)),
        ],
        out_specs=[
            pl.BlockSpec((BN, OUT), lambda i: (i, 0)),
            pl.BlockSpec((BN, 2 * H), lambda i: (i, 0)),
        ],
        out_shape=[
            jax.ShapeDtypeStruct((N, OUT), jnp.float32),
            jax.ShapeDtypeStruct((N, 2 * H), jnp.float32),
        ],
    )(x, state, w1, w2, at)

    src2 = edge_index[0].reshape(E // C, C)
    dst2 = edge_index[1].reshape(E // C, C)

    sc = functools.partial(
        pl.kernel,
        mesh=plsc.VectorSubcoreMesh(core_axis_name="c", subcore_axis_name="s"),
        compiler_params=pltpu.CompilerParams(
            use_tc_tiling_on_sc=False, needs_layout_passes=False),
        out_type=jax.ShapeDtypeStruct((NC, N, ACCW), jnp.float32),
        scratch_types=[
            pltpu.VMEM_SHARED((N, ACCW), jnp.float32),
            pltpu.VMEM((CB, C), jnp.int32),
            pltpu.VMEM((CB, C), jnp.int32),
            pltpu.VMEM((C, 2 * H), jnp.float32),
            pltpu.VMEM((C, 2 * H), jnp.float32),
            pltpu.VMEM((C, OUT), jnp.float32),
            pltpu.VMEM((C, ACCW), jnp.float32),
            pltpu.VMEM((5, ACCW), jnp.float32),
            pltpu.SemaphoreType.DMA,
        ],
    )(_sc_body)
    parts = sc(h, stab, src2, dst2)

    out = pl.pallas_call(
        _tc_tail,
        grid=(N // BN,),
        in_specs=[
            pl.BlockSpec((NC, BN, ACCW), lambda i: (0, i, 0)),
            pl.BlockSpec((BN, OUT), lambda i: (i, 0)),
        ],
        out_specs=pl.BlockSpec((BN, OUT), lambda i: (i, 0)),
        out_shape=jax.ShapeDtypeStruct((N, OUT), jnp.float32),
    )(parts, h)
    return out


# speedup vs baseline: 12.2966x; 1.2028x over previous
import functools

import jax
import jax.numpy as jnp
from jax import lax
from jax.experimental import pallas as pl
from jax.experimental.pallas import tpu as pltpu
from jax.experimental.pallas import tpu_sc as plsc

N = 10000
E = 320000
D = 128
OUT = 128
H = 4
HD = 32
NEG_SLOPE = 0.2

NC = 2
NS = 16
NW = NC * NS
ET = E // NW
C = 80
NCHUNK = ET // C
NT = N // NS
DW = 8
BN = 1000


def _tc_head(x_ref, st_ref, w1_ref, w2_ref, at_ref, h_ref, s_ref):
    h = jnp.dot(x_ref[...], w1_ref[...], preferred_element_type=jnp.float32)
    h = h + jnp.dot(st_ref[...], w2_ref[...], preferred_element_type=jnp.float32)
    h_ref[...] = h
    s_ref[...] = jnp.dot(h, at_ref[...], preferred_element_type=jnp.float32)


def _tc_tail(p_ref, d_ref, h_ref, o_ref):
    acc = p_ref[0] + p_ref[1]
    den = d_ref[0] + d_ref[1]
    cols = []
    for hh in range(H):
        num = acc[:, hh * HD:(hh + 1) * HD]
        cols.append(num / (den[:, hh][:, None] + 1e-12))
    o = jnp.concatenate(cols, axis=1) + h_ref[...]
    o_ref[...] = jnp.where(o > 0, o, jnp.exp(jnp.minimum(o, 0.0)) - 1.0)


def _splat(val):
    return jnp.full((16,), val, jnp.int32)


def _sc_body(h_hbm, stab_hbm, src_hbm, dst_hbm, acc_out, den_out,
             acc_sh, den_sh, sidx, didx, ssrc, sdst, hrows, msg, wmsg,
             zbuf, zbufd, sem_i, sem_g, sem_s):
    c = lax.axis_index("c")
    s = lax.axis_index("s")
    gid = c * NS + s
    row0 = gid * NCHUNK
    zero16 = jnp.zeros((16,), jnp.float32)

    for r in range(5):
        for k in range(OUT // 16):
            zbuf[r, pl.ds(k * 16, 16)] = zero16
    for col in range(DW):
        plsc.store_scatter(zbufd, [lax.iota(jnp.int32, 16), _splat(col)],
                           zero16)
    for p in range(2):
        for g in range(C // 16):
            ev = lax.iota(jnp.int32, 16) + g * 16
            for col in range(H, DW):
                plsc.store_scatter(wmsg.at[p], [ev, _splat(col)], zero16)

    @pl.loop(0, NT // 5)
    def _zero(i):
        pltpu.sync_copy(zbuf, acc_sh.at[pl.ds(s * NT + i * 5, 5)])

    @pl.loop(0, NT // 5 // 5)
    def _zerod(i):
        pltpu.sync_copy(zbufd.at[pl.ds(0, 16)],
                        den_sh.at[pl.ds(s * NT + i * 25, 16)])
        pltpu.sync_copy(zbufd.at[pl.ds(0, 9)],
                        den_sh.at[pl.ds(s * NT + i * 25 + 16, 9)])

    plsc.subcore_barrier()

    def _issue_idx(j):
        pltpu.sync_copy(src_hbm.at[pl.ds(row0 + j, 1)],
                        sidx.at[lax.rem(j, 2)])
        pltpu.sync_copy(dst_hbm.at[pl.ds(row0 + j, 1)],
                        didx.at[lax.rem(j, 4)])

    def _issue_gathers(j):
        p = lax.rem(j, 2)
        pltpu.async_copy(stab_hbm.at[sidx.at[p, 0]], ssrc, sem_g)
        pltpu.async_copy(stab_hbm.at[didx.at[lax.rem(j, 4), 0]], sdst, sem_g)
        pltpu.async_copy(h_hbm.at[sidx.at[p, 0]], hrows.at[p], sem_g)

    _issue_idx(0)
    _issue_gathers(0)

    @pl.loop(0, NCHUNK)
    def _chunk(j):
        p = lax.rem(j, 2)
        r4 = lax.rem(j, 4)
        pltpu.make_async_copy(stab_hbm.at[sidx.at[p, 0]], ssrc, sem_g).wait()
        pltpu.make_async_copy(stab_hbm.at[didx.at[r4, 0]], sdst, sem_g).wait()
        pltpu.make_async_copy(h_hbm.at[sidx.at[p, 0]], hrows.at[p],
                              sem_g).wait()
        @pl.when(j >= 2)
        def _():
            r4p = lax.rem(j + 2, 4)
            pltpu.make_async_copy(msg.at[p], acc_sh.at[didx.at[r4p, 0]],
                                  sem_s).wait()
            pltpu.make_async_copy(wmsg.at[p], den_sh.at[didx.at[r4p, 0]],
                                  sem_s).wait()

        for g in range(C // 16):
            ev = lax.iota(jnp.int32, 16) + g * 16
            for hh in range(H):
                s1 = plsc.load_gather(ssrc, [ev, _splat(hh)])
                s2 = plsc.load_gather(sdst, [ev, _splat(4 + hh)])
                logit = s1 + s2
                logit = jnp.where(logit >= 0, logit, NEG_SLOPE * logit)
                plsc.store_scatter(wmsg.at[p], [ev, _splat(hh)],
                                   jnp.exp(logit))

        @pl.when(j < NCHUNK - 1)
        def _():
            _issue_idx(j + 1)
            _issue_gathers(j + 1)

        @pl.loop(0, C, unroll=2)
        def _row(i):
            ii = jnp.zeros((16,), jnp.int32) + i
            for hh in range(H):
                wv = plsc.load_gather(wmsg.at[p], [ii, _splat(hh)])
                for q in range(2):
                    off = hh * HD + q * 16
                    msg[p, i, pl.ds(off, 16)] = (
                        hrows[p, i, pl.ds(off, 16)] * wv)

        pltpu.async_copy(msg.at[p], acc_sh.at[didx.at[r4, 0]], sem_s,
                         add=True)
        pltpu.async_copy(wmsg.at[p], den_sh.at[didx.at[r4, 0]], sem_s,
                         add=True)

    for j in (NCHUNK - 2, NCHUNK - 1):
        p = j % 2
        r4 = j % 4
        pltpu.make_async_copy(msg.at[p], acc_sh.at[didx.at[r4, 0]],
                              sem_s).wait()
        pltpu.make_async_copy(wmsg.at[p], den_sh.at[didx.at[r4, 0]],
                              sem_s).wait()

    plsc.subcore_barrier()
    pltpu.sync_copy(acc_sh.at[pl.ds(s * NT, NT)],
                    acc_out.at[c, pl.ds(s * NT, NT)])
    pltpu.sync_copy(den_sh.at[pl.ds(s * NT, NT)],
                    den_out.at[c, pl.ds(s * NT, NT)])


@jax.jit
def kernel(x, state, edge_index, edge_weight, W, attn_src, attn_dst):
    del edge_weight
    eye = jnp.eye(H, dtype=jnp.float32)
    a_src = (attn_src[:, :, None] * eye[:, None, :]).reshape(OUT, H)
    a_dst = (attn_dst[:, :, None] * eye[:, None, :]).reshape(OUT, H)
    at = jnp.concatenate([a_src, a_dst], axis=1)
    w1 = W[:D]
    w2 = W[D:]

    h, stab = pl.pallas_call(
        _tc_head,
        grid=(N // BN,),
        in_specs=[
            pl.BlockSpec((BN, D), lambda i: (i, 0)),
            pl.BlockSpec((BN, D), lambda i: (i, 0)),
            pl.BlockSpec((D, OUT), lambda i: (0, 0)),
            pl.BlockSpec((D, OUT), lambda i: (0, 0)),
            pl.BlockSpec((OUT, 2 * H), lambda i: (0, 0)),
        ],
        out_specs=[
            pl.BlockSpec((BN, OUT), lambda i: (i, 0)),
            pl.BlockSpec((BN, 2 * H), lambda i: (i, 0)),
        ],
        out_shape=[
            jax.ShapeDtypeStruct((N, OUT), jnp.float32),
            jax.ShapeDtypeStruct((N, 2 * H), jnp.float32),
        ],
    )(x, state, w1, w2, at)

    src2 = edge_index[0].reshape(E // C, C)
    dst2 = edge_index[1].reshape(E // C, C)

    sc = functools.partial(
        pl.kernel,
        mesh=plsc.VectorSubcoreMesh(core_axis_name="c", subcore_axis_name="s"),
        compiler_params=pltpu.CompilerParams(
            use_tc_tiling_on_sc=False, needs_layout_passes=False),
        out_type=[
            jax.ShapeDtypeStruct((NC, N, OUT), jnp.float32),
            jax.ShapeDtypeStruct((NC, N, DW), jnp.float32),
        ],
        scratch_types=[
            pltpu.VMEM_SHARED((N, OUT), jnp.float32),
            pltpu.VMEM_SHARED((N, DW), jnp.float32),
            pltpu.VMEM((2, 1, C), jnp.int32),
            pltpu.VMEM((4, 1, C), jnp.int32),
            pltpu.VMEM((C, 2 * H), jnp.float32),
            pltpu.VMEM((C, 2 * H), jnp.float32),
            pltpu.VMEM((2, C, OUT), jnp.float32),
            pltpu.VMEM((2, C, OUT), jnp.float32),
            pltpu.VMEM((2, C, DW), jnp.float32),
            pltpu.VMEM((5, OUT), jnp.float32),
            pltpu.VMEM((16, DW), jnp.float32),
            pltpu.SemaphoreType.DMA,
            pltpu.SemaphoreType.DMA,
            pltpu.SemaphoreType.DMA,
        ],
    )(_sc_body)
    parts, dens = sc(h, stab, src2, dst2)

    out = pl.pallas_call(
        _tc_tail,
        grid=(N // BN,),
        in_specs=[
            pl.BlockSpec((NC, BN, OUT), lambda i: (0, i, 0)),
            pl.BlockSpec((NC, BN, DW), lambda i: (0, i, 0)),
            pl.BlockSpec((BN, OUT), lambda i: (i, 0)),
        ],
        out_specs=pl.BlockSpec((BN, OUT), lambda i: (i, 0)),
        out_shape=jax.ShapeDtypeStruct((N, OUT), jnp.float32),
    )(parts, dens, h)
    return out


# speedup vs baseline: 12.3423x; 1.0037x over previous
import functools

import jax
import jax.numpy as jnp
from jax import lax
from jax.experimental import pallas as pl
from jax.experimental.pallas import tpu as pltpu
from jax.experimental.pallas import tpu_sc as plsc

N = 10000
E = 320000
D = 128
OUT = 128
H = 4
HD = 32
NEG_SLOPE = 0.2

NC = 2
NS = 16
NW = NC * NS
ET = E // NW
C = 80
NCHUNK = ET // C
NT = N // NS
DW = 8
BN = 1000


def _tc_head(x_ref, st_ref, w1_ref, w2_ref, at_ref, h_ref, s_ref):
    h = jnp.dot(x_ref[...], w1_ref[...], preferred_element_type=jnp.float32)
    h = h + jnp.dot(st_ref[...], w2_ref[...], preferred_element_type=jnp.float32)
    h_ref[...] = h
    s_ref[...] = jnp.dot(h, at_ref[...], preferred_element_type=jnp.float32)


def _tc_tail(p_ref, d_ref, h_ref, o_ref):
    acc = p_ref[0] + p_ref[1]
    den = d_ref[0] + d_ref[1]
    cols = []
    for hh in range(H):
        num = acc[:, hh * HD:(hh + 1) * HD]
        cols.append(num / (den[:, hh][:, None] + 1e-12))
    o = jnp.concatenate(cols, axis=1) + h_ref[...]
    o_ref[...] = jnp.where(o > 0, o, jnp.exp(jnp.minimum(o, 0.0)) - 1.0)


def _splat(val):
    return jnp.full((16,), val, jnp.int32)


def _sc_body(h_hbm, stab_hbm, src_hbm, dst_hbm, acc_out, den_out,
             acc_sh, den_sh, sidx, didx, ssrc, sdst, hrows, msg, wmsg,
             zbuf, zbufd, sem_i, sem_g, sem_s):
    c = lax.axis_index("c")
    s = lax.axis_index("s")
    gid = c * NS + s
    row0 = gid * NCHUNK
    zero16 = jnp.zeros((16,), jnp.float32)

    for r in range(5):
        for k in range(OUT // 16):
            zbuf[r, pl.ds(k * 16, 16)] = zero16
    for col in range(DW):
        plsc.store_scatter(zbufd, [lax.iota(jnp.int32, 16), _splat(col)],
                           zero16)
    for p in range(2):
        for g in range(C // 16):
            ev = lax.iota(jnp.int32, 16) + g * 16
            for col in range(H, DW):
                plsc.store_scatter(wmsg.at[p], [ev, _splat(col)], zero16)

    @pl.loop(0, NT // 5)
    def _zero(i):
        pltpu.sync_copy(zbuf, acc_sh.at[pl.ds(s * NT + i * 5, 5)])

    @pl.loop(0, NT // 5 // 5)
    def _zerod(i):
        pltpu.sync_copy(zbufd.at[pl.ds(0, 16)],
                        den_sh.at[pl.ds(s * NT + i * 25, 16)])
        pltpu.sync_copy(zbufd.at[pl.ds(0, 9)],
                        den_sh.at[pl.ds(s * NT + i * 25 + 16, 9)])

    plsc.subcore_barrier()

    def _issue_idx(j):
        pltpu.sync_copy(src_hbm.at[pl.ds(row0 + j, 1)],
                        sidx.at[lax.rem(j, 2)])
        pltpu.sync_copy(dst_hbm.at[pl.ds(row0 + j, 1)],
                        didx.at[lax.rem(j, 4)])

    def _issue_gathers(j):
        p = lax.rem(j, 2)
        pltpu.async_copy(stab_hbm.at[sidx.at[p, 0]], ssrc, sem_g)
        pltpu.async_copy(stab_hbm.at[didx.at[lax.rem(j, 4), 0]], sdst, sem_g)
        pltpu.async_copy(h_hbm.at[sidx.at[p, 0]], hrows.at[p], sem_g)

    _issue_idx(0)
    _issue_gathers(0)

    @pl.loop(0, NCHUNK)
    def _chunk(j):
        p = lax.rem(j, 2)
        r4 = lax.rem(j, 4)
        pltpu.make_async_copy(stab_hbm.at[sidx.at[p, 0]], ssrc, sem_g).wait()
        pltpu.make_async_copy(stab_hbm.at[didx.at[r4, 0]], sdst, sem_g).wait()
        pltpu.make_async_copy(h_hbm.at[sidx.at[p, 0]], hrows.at[p],
                              sem_g).wait()
        @pl.when(j >= 2)
        def _():
            r4p = lax.rem(j + 2, 4)
            pltpu.make_async_copy(msg.at[p], acc_sh.at[didx.at[r4p, 0]],
                                  sem_s).wait()
            pltpu.make_async_copy(wmsg.at[p], den_sh.at[didx.at[r4p, 0]],
                                  sem_s).wait()

        for g in range(C // 16):
            ev = lax.iota(jnp.int32, 16) + g * 16
            for hh in range(H):
                s1 = plsc.load_gather(ssrc, [ev, _splat(hh)])
                s2 = plsc.load_gather(sdst, [ev, _splat(4 + hh)])
                logit = s1 + s2
                logit = jnp.where(logit >= 0, logit, NEG_SLOPE * logit)
                plsc.store_scatter(wmsg.at[p], [ev, _splat(hh)],
                                   jnp.exp(logit))

        @pl.when(j < NCHUNK - 1)
        def _():
            _issue_idx(j + 1)
            _issue_gathers(j + 1)

        @pl.loop(0, C, unroll=4)
        def _row(i):
            ii = jnp.zeros((16,), jnp.int32) + i
            for hh in range(H):
                wv = plsc.load_gather(wmsg.at[p], [ii, _splat(hh)])
                for q in range(2):
                    off = hh * HD + q * 16
                    msg[p, i, pl.ds(off, 16)] = (
                        hrows[p, i, pl.ds(off, 16)] * wv)

        pltpu.async_copy(msg.at[p], acc_sh.at[didx.at[r4, 0]], sem_s,
                         add=True)
        pltpu.async_copy(wmsg.at[p], den_sh.at[didx.at[r4, 0]], sem_s,
                         add=True)

    for j in (NCHUNK - 2, NCHUNK - 1):
        p = j % 2
        r4 = j % 4
        pltpu.make_async_copy(msg.at[p], acc_sh.at[didx.at[r4, 0]],
                              sem_s).wait()
        pltpu.make_async_copy(wmsg.at[p], den_sh.at[didx.at[r4, 0]],
                              sem_s).wait()

    plsc.subcore_barrier()
    pltpu.sync_copy(acc_sh.at[pl.ds(s * NT, NT)],
                    acc_out.at[c, pl.ds(s * NT, NT)])
    pltpu.sync_copy(den_sh.at[pl.ds(s * NT, NT)],
                    den_out.at[c, pl.ds(s * NT, NT)])


@jax.jit
def kernel(x, state, edge_index, edge_weight, W, attn_src, attn_dst):
    del edge_weight
    eye = jnp.eye(H, dtype=jnp.float32)
    a_src = (attn_src[:, :, None] * eye[:, None, :]).reshape(OUT, H)
    a_dst = (attn_dst[:, :, None] * eye[:, None, :]).reshape(OUT, H)
    at = jnp.concatenate([a_src, a_dst], axis=1)
    w1 = W[:D]
    w2 = W[D:]

    h, stab = pl.pallas_call(
        _tc_head,
        grid=(N // BN,),
        in_specs=[
            pl.BlockSpec((BN, D), lambda i: (i, 0)),
            pl.BlockSpec((BN, D), lambda i: (i, 0)),
            pl.BlockSpec((D, OUT), lambda i: (0, 0)),
            pl.BlockSpec((D, OUT), lambda i: (0, 0)),
            pl.BlockSpec((OUT, 2 * H), lambda i: (0, 0)),
        ],
        out_specs=[
            pl.BlockSpec((BN, OUT), lambda i: (i, 0)),
            pl.BlockSpec((BN, 2 * H), lambda i: (i, 0)),
        ],
        out_shape=[
            jax.ShapeDtypeStruct((N, OUT), jnp.float32),
            jax.ShapeDtypeStruct((N, 2 * H), jnp.float32),
        ],
    )(x, state, w1, w2, at)

    src2 = edge_index[0].reshape(E // C, C)
    dst2 = edge_index[1].reshape(E // C, C)

    sc = functools.partial(
        pl.kernel,
        mesh=plsc.VectorSubcoreMesh(core_axis_name="c", subcore_axis_name="s"),
        compiler_params=pltpu.CompilerParams(
            use_tc_tiling_on_sc=False, needs_layout_passes=False),
        out_type=[
            jax.ShapeDtypeStruct((NC, N, OUT), jnp.float32),
            jax.ShapeDtypeStruct((NC, N, DW), jnp.float32),
        ],
        scratch_types=[
            pltpu.VMEM_SHARED((N, OUT), jnp.float32),
            pltpu.VMEM_SHARED((N, DW), jnp.float32),
            pltpu.VMEM((2, 1, C), jnp.int32),
            pltpu.VMEM((4, 1, C), jnp.int32),
            pltpu.VMEM((C, 2 * H), jnp.float32),
            pltpu.VMEM((C, 2 * H), jnp.float32),
            pltpu.VMEM((2, C, OUT), jnp.float32),
            pltpu.VMEM((2, C, OUT), jnp.float32),
            pltpu.VMEM((2, C, DW), jnp.float32),
            pltpu.VMEM((5, OUT), jnp.float32),
            pltpu.VMEM((16, DW), jnp.float32),
            pltpu.SemaphoreType.DMA,
            pltpu.SemaphoreType.DMA,
            pltpu.SemaphoreType.DMA,
        ],
    )(_sc_body)
    parts, dens = sc(h, stab, src2, dst2)

    out = pl.pallas_call(
        _tc_tail,
        grid=(N // BN,),
        in_specs=[
            pl.BlockSpec((NC, BN, OUT), lambda i: (0, i, 0)),
            pl.BlockSpec((NC, BN, DW), lambda i: (0, i, 0)),
            pl.BlockSpec((BN, OUT), lambda i: (i, 0)),
        ],
        out_specs=pl.BlockSpec((BN, OUT), lambda i: (i, 0)),
        out_shape=jax.ShapeDtypeStruct((N, OUT), jnp.float32),
    )(parts, dens, h)
    return out


# speedup vs baseline: 14.8380x; 1.2022x over previous
import functools

import jax
import jax.numpy as jnp
from jax import lax
from jax.experimental import pallas as pl
from jax.experimental.pallas import tpu as pltpu
from jax.experimental.pallas import tpu_sc as plsc

N = 10000
E = 320000
D = 128
OUT = 128
H = 4
HD = 32
NEG_SLOPE = 0.2

NC = 2
NS = 16
NW = NC * NS
ET = E // NW
C = 80
NCHUNK = ET // C
NT = N // NS
DW = 8
BN = 1000


def _tc_head(x_ref, st_ref, w1_ref, w2_ref, at_ref, h_ref, s_ref):
    h = jnp.dot(x_ref[...], w1_ref[...], preferred_element_type=jnp.float32)
    h = h + jnp.dot(st_ref[...], w2_ref[...], preferred_element_type=jnp.float32)
    h_ref[...] = h
    s_ref[...] = jnp.dot(h, at_ref[...], preferred_element_type=jnp.float32)


def _tc_tail(p_ref, d_ref, h_ref, o_ref):
    acc = p_ref[0] + p_ref[1]
    den = d_ref[0] + d_ref[1]
    cols = []
    for hh in range(H):
        num = acc[:, hh * HD:(hh + 1) * HD]
        cols.append(num / (den[:, hh][:, None] + 1e-12))
    o = jnp.concatenate(cols, axis=1) + h_ref[...]
    o_ref[...] = jnp.where(o > 0, o, jnp.exp(jnp.minimum(o, 0.0)) - 1.0)


def _splat(val):
    return jnp.full((16,), val, jnp.int32)


def _sc_body(h_hbm, stab_hbm, src_hbm, dst_hbm, acc_out, den_out,
             acc_sh, den_sh, sidx, didx, ssrc, sdst, hrows, msg, wmsg,
             zbuf, zbufd, sem_i, sem_g, sem_s):
    c = lax.axis_index("c")
    s = lax.axis_index("s")
    gid = c * NS + s
    row0 = gid * NCHUNK
    zero16 = jnp.zeros((16,), jnp.float32)

    for r in range(5):
        for k in range(OUT // 16):
            zbuf[r, pl.ds(k * 16, 16)] = zero16
    for col in range(DW):
        plsc.store_scatter(zbufd, [lax.iota(jnp.int32, 16), _splat(col)],
                           zero16)
    for p in range(2):
        for g in range(C // 16):
            ev = lax.iota(jnp.int32, 16) + g * 16
            for col in range(H, DW):
                plsc.store_scatter(wmsg.at[p], [ev, _splat(col)], zero16)

    @pl.loop(0, NT // 5)
    def _zero(i):
        pltpu.sync_copy(zbuf, acc_sh.at[pl.ds(s * NT + i * 5, 5)])

    @pl.loop(0, NT // 5 // 5)
    def _zerod(i):
        pltpu.sync_copy(zbufd.at[pl.ds(0, 16)],
                        den_sh.at[pl.ds(s * NT + i * 25, 16)])
        pltpu.sync_copy(zbufd.at[pl.ds(0, 9)],
                        den_sh.at[pl.ds(s * NT + i * 25 + 16, 9)])

    plsc.subcore_barrier()

    def _issue_idx(j):
        pltpu.async_copy(src_hbm.at[pl.ds(row0 + j, 1)],
                         sidx.at[lax.rem(j, 4)], sem_i)
        pltpu.async_copy(dst_hbm.at[pl.ds(row0 + j, 1)],
                         didx.at[lax.rem(j, 4)], sem_i)

    def _wait_idx(j):
        pltpu.make_async_copy(src_hbm.at[pl.ds(row0 + j, 1)],
                              sidx.at[lax.rem(j, 4)], sem_i).wait()
        pltpu.make_async_copy(dst_hbm.at[pl.ds(row0 + j, 1)],
                              didx.at[lax.rem(j, 4)], sem_i).wait()

    def _issue_gathers(j):
        p = lax.rem(j, 2)
        r4 = lax.rem(j, 4)
        pltpu.async_copy(stab_hbm.at[sidx.at[r4, 0]], ssrc, sem_g)
        pltpu.async_copy(stab_hbm.at[didx.at[r4, 0]], sdst, sem_g)
        pltpu.async_copy(h_hbm.at[sidx.at[r4, 0]], hrows.at[p], sem_g)

    _issue_idx(0)
    _issue_idx(1)
    _wait_idx(0)
    _issue_gathers(0)

    @pl.loop(0, NCHUNK)
    def _chunk(j):
        p = lax.rem(j, 2)
        r4 = lax.rem(j, 4)
        pltpu.make_async_copy(stab_hbm.at[sidx.at[r4, 0]], ssrc, sem_g).wait()
        pltpu.make_async_copy(stab_hbm.at[didx.at[r4, 0]], sdst, sem_g).wait()
        pltpu.make_async_copy(h_hbm.at[sidx.at[r4, 0]], hrows.at[p],
                              sem_g).wait()
        @pl.when(j >= 2)
        def _():
            r4p = lax.rem(j + 2, 4)
            pltpu.make_async_copy(msg.at[p], acc_sh.at[didx.at[r4p, 0]],
                                  sem_s).wait()
            pltpu.make_async_copy(wmsg.at[p], den_sh.at[didx.at[r4p, 0]],
                                  sem_s).wait()

        for g in range(C // 16):
            ev = lax.iota(jnp.int32, 16) + g * 16
            for hh in range(H):
                s1 = plsc.load_gather(ssrc, [ev, _splat(hh)])
                s2 = plsc.load_gather(sdst, [ev, _splat(4 + hh)])
                logit = s1 + s2
                logit = jnp.where(logit >= 0, logit, NEG_SLOPE * logit)
                plsc.store_scatter(wmsg.at[p], [ev, _splat(hh)],
                                   jnp.exp(logit))

        @pl.when(j < NCHUNK - 1)
        def _():
            _wait_idx(j + 1)
            _issue_gathers(j + 1)

        @pl.when(j < NCHUNK - 2)
        def _():
            _issue_idx(j + 2)

        @pl.loop(0, C, unroll=4)
        def _row(i):
            ii = jnp.zeros((16,), jnp.int32) + i
            for hh in range(H):
                wv = plsc.load_gather(wmsg.at[p], [ii, _splat(hh)])
                for q in range(2):
                    off = hh * HD + q * 16
                    msg[p, i, pl.ds(off, 16)] = (
                        hrows[p, i, pl.ds(off, 16)] * wv)

        pltpu.async_copy(msg.at[p], acc_sh.at[didx.at[r4, 0]], sem_s,
                         add=True)
        pltpu.async_copy(wmsg.at[p], den_sh.at[didx.at[r4, 0]], sem_s,
                         add=True)

    for j in (NCHUNK - 2, NCHUNK - 1):
        p = j % 2
        r4 = j % 4
        pltpu.make_async_copy(msg.at[p], acc_sh.at[didx.at[r4, 0]],
                              sem_s).wait()
        pltpu.make_async_copy(wmsg.at[p], den_sh.at[didx.at[r4, 0]],
                              sem_s).wait()

    plsc.subcore_barrier()
    pltpu.sync_copy(acc_sh.at[pl.ds(s * NT, NT)],
                    acc_out.at[c, pl.ds(s * NT, NT)])
    pltpu.sync_copy(den_sh.at[pl.ds(s * NT, NT)],
                    den_out.at[c, pl.ds(s * NT, NT)])


@jax.jit
def kernel(x, state, edge_index, edge_weight, W, attn_src, attn_dst):
    del edge_weight
    eye = jnp.eye(H, dtype=jnp.float32)
    a_src = (attn_src[:, :, None] * eye[:, None, :]).reshape(OUT, H)
    a_dst = (attn_dst[:, :, None] * eye[:, None, :]).reshape(OUT, H)
    at = jnp.concatenate([a_src, a_dst], axis=1)
    w1 = W[:D]
    w2 = W[D:]

    h, stab = pl.pallas_call(
        _tc_head,
        grid=(N // BN,),
        in_specs=[
            pl.BlockSpec((BN, D), lambda i: (i, 0)),
            pl.BlockSpec((BN, D), lambda i: (i, 0)),
            pl.BlockSpec((D, OUT), lambda i: (0, 0)),
            pl.BlockSpec((D, OUT), lambda i: (0, 0)),
            pl.BlockSpec((OUT, 2 * H), lambda i: (0, 0)),
        ],
        out_specs=[
            pl.BlockSpec((BN, OUT), lambda i: (i, 0)),
            pl.BlockSpec((BN, 2 * H), lambda i: (i, 0)),
        ],
        out_shape=[
            jax.ShapeDtypeStruct((N, OUT), jnp.float32),
            jax.ShapeDtypeStruct((N, 2 * H), jnp.float32),
        ],
    )(x, state, w1, w2, at)

    src2 = edge_index[0].reshape(E // C, C)
    dst2 = edge_index[1].reshape(E // C, C)

    sc = functools.partial(
        pl.kernel,
        mesh=plsc.VectorSubcoreMesh(core_axis_name="c", subcore_axis_name="s"),
        compiler_params=pltpu.CompilerParams(
            use_tc_tiling_on_sc=False, needs_layout_passes=False),
        out_type=[
            jax.ShapeDtypeStruct((NC, N, OUT), jnp.float32),
            jax.ShapeDtypeStruct((NC, N, DW), jnp.float32),
        ],
        scratch_types=[
            pltpu.VMEM_SHARED((N, OUT), jnp.float32),
            pltpu.VMEM_SHARED((N, DW), jnp.float32),
            pltpu.VMEM((4, 1, C), jnp.int32),
            pltpu.VMEM((4, 1, C), jnp.int32),
            pltpu.VMEM((C, 2 * H), jnp.float32),
            pltpu.VMEM((C, 2 * H), jnp.float32),
            pltpu.VMEM((2, C, OUT), jnp.float32),
            pltpu.VMEM((2, C, OUT), jnp.float32),
            pltpu.VMEM((2, C, DW), jnp.float32),
            pltpu.VMEM((5, OUT), jnp.float32),
            pltpu.VMEM((16, DW), jnp.float32),
            pltpu.SemaphoreType.DMA,
            pltpu.SemaphoreType.DMA,
            pltpu.SemaphoreType.DMA,
        ],
    )(_sc_body)
    parts, dens = sc(h, stab, src2, dst2)

    out = pl.pallas_call(
        _tc_tail,
        grid=(N // BN,),
        in_specs=[
            pl.BlockSpec((NC, BN, OUT), lambda i: (0, i, 0)),
            pl.BlockSpec((NC, BN, DW), lambda i: (0, i, 0)),
            pl.BlockSpec((BN, OUT), lambda i: (i, 0)),
        ],
        out_specs=pl.BlockSpec((BN, OUT), lambda i: (i, 0)),
        out_shape=jax.ShapeDtypeStruct((N, OUT), jnp.float32),
    )(parts, dens, h)
    return out
